# baseline (speedup 1.0000x reference)
import jax
import jax.numpy as jnp
from jax import lax
from jax.experimental import pallas as pl
from jax.experimental.pallas import tpu as pltpu

_INPUT_DIM = 2048
_HIDDEN_DIM = 512
_NUM_EXPERTS = 64
_N_TOKENS = 8192
_BLK = 2048


def _gating_kernel(xa_ref, xb_ref, w1_ref, b1_ref, w2_ref, b2_ref,
                   idx_ref, wgt_ref):
    half = _INPUT_DIM // 2
    ha = jnp.dot(xa_ref[...], w1_ref[pl.ds(0, half), :],
                 preferred_element_type=jnp.float32)
    hb = jnp.dot(xb_ref[...], w1_ref[pl.ds(half, half), :],
                 preferred_element_type=jnp.float32)
    h = jnp.maximum(ha + hb + b1_ref[...], 0.0)
    logits = jnp.dot(h, w2_ref[...], preferred_element_type=jnp.float32)
    logits = logits + b2_ref[...]

    ids = lax.broadcasted_iota(jnp.int32, logits.shape, 1).astype(jnp.float32)
    neg_inf = jnp.float32(-jnp.inf)
    big = jnp.float32(_NUM_EXPERTS)

    m1 = jnp.max(logits, axis=1, keepdims=True)
    i1 = jnp.min(jnp.where(logits == m1, ids, big), axis=1, keepdims=True)
    masked = jnp.where(ids == i1, neg_inf, logits)
    m2 = jnp.max(masked, axis=1, keepdims=True)
    i2 = jnp.min(jnp.where(masked == m2, ids, big), axis=1, keepdims=True)

    e2 = jnp.exp(m2 - m1)
    w1v = 1.0 / (1.0 + e2)
    w2v = e2 * w1v

    idx_ref[...] = jnp.concatenate([i1, i2], axis=1).astype(jnp.int32)
    wgt_ref[...] = jnp.concatenate([w1v, w2v], axis=1)


def kernel(x, W1, b1, W2, b2):
    n_blocks = _N_TOKENS // _BLK
    b1r = b1.reshape(1, _HIDDEN_DIM)
    b2r = b2.reshape(1, _NUM_EXPERTS)

    indices, weights = pl.pallas_call(
        _gating_kernel,
        grid=(n_blocks,),
        in_specs=[
            pl.BlockSpec((_BLK, _INPUT_DIM // 2), lambda i: (i, 0)),
            pl.BlockSpec((_BLK, _INPUT_DIM // 2), lambda i: (i, 1)),
            pl.BlockSpec((_INPUT_DIM, _HIDDEN_DIM), lambda i: (0, 0)),
            pl.BlockSpec((1, _HIDDEN_DIM), lambda i: (0, 0)),
            pl.BlockSpec((_HIDDEN_DIM, _NUM_EXPERTS), lambda i: (0, 0)),
            pl.BlockSpec((1, _NUM_EXPERTS), lambda i: (0, 0)),
        ],
        out_specs=[
            pl.BlockSpec((_BLK, 2), lambda i: (i, 0)),
            pl.BlockSpec((_BLK, 2), lambda i: (i, 0)),
        ],
        out_shape=[
            jax.ShapeDtypeStruct((_N_TOKENS, 2), jnp.int32),
            jax.ShapeDtypeStruct((_N_TOKENS, 2), jnp.float32),
        ],
        compiler_params=pltpu.CompilerParams(
            dimension_semantics=("arbitrary",),
            vmem_limit_bytes=100 * 1024 * 1024,
        ),
    )(x, x, W1, b1r, W2, b2r)
    return (indices, weights)


# speedup vs baseline: 1.0002x; 1.0002x over previous
import jax
import jax.numpy as jnp
from jax import lax
from jax.experimental import pallas as pl
from jax.experimental.pallas import tpu as pltpu

_INPUT_DIM = 2048
_HIDDEN_DIM = 512
_NUM_EXPERTS = 64
_N_TOKENS = 8192
_BLK = 2048


def _gating_kernel(x_ref, w1_ref, b1_ref, w2_ref, b2_ref, idx_ref, wgt_ref):
    x = x_ref[...]
    h = jnp.dot(x, w1_ref[...], preferred_element_type=jnp.float32)
    h = jnp.maximum(h + b1_ref[...], 0.0)
    logits = jnp.dot(h, w2_ref[...], preferred_element_type=jnp.float32)
    logits = logits + b2_ref[...]

    ids = lax.broadcasted_iota(jnp.int32, logits.shape, 1).astype(jnp.float32)
    neg_inf = jnp.float32(-jnp.inf)
    big = jnp.float32(_NUM_EXPERTS)

    m1 = jnp.max(logits, axis=1, keepdims=True)
    i1 = jnp.min(jnp.where(logits == m1, ids, big), axis=1, keepdims=True)
    masked = jnp.where(ids == i1, neg_inf, logits)
    m2 = jnp.max(masked, axis=1, keepdims=True)
    i2 = jnp.min(jnp.where(masked == m2, ids, big), axis=1, keepdims=True)

    e2 = jnp.exp(m2 - m1)
    w1v = 1.0 / (1.0 + e2)
    w2v = e2 * w1v

    idx_ref[...] = jnp.concatenate([i1, i2], axis=1).astype(jnp.int32)
    wgt_ref[...] = jnp.concatenate([w1v, w2v], axis=1)


def kernel(x, W1, b1, W2, b2):
    n_blocks = _N_TOKENS // _BLK
    b1r = b1.reshape(1, _HIDDEN_DIM)
    b2r = b2.reshape(1, _NUM_EXPERTS)

    indices, weights = pl.pallas_call(
        _gating_kernel,
        grid=(n_blocks,),
        in_specs=[
            pl.BlockSpec((_BLK, _INPUT_DIM), lambda i: (i, 0)),
            pl.BlockSpec((_INPUT_DIM, _HIDDEN_DIM), lambda i: (0, 0)),
            pl.BlockSpec((1, _HIDDEN_DIM), lambda i: (0, 0)),
            pl.BlockSpec((_HIDDEN_DIM, _NUM_EXPERTS), lambda i: (0, 0)),
            pl.BlockSpec((1, _NUM_EXPERTS), lambda i: (0, 0)),
        ],
        out_specs=[
            pl.BlockSpec((_BLK, 2), lambda i: (i, 0)),
            pl.BlockSpec((_BLK, 2), lambda i: (i, 0)),
        ],
        out_shape=[
            jax.ShapeDtypeStruct((_N_TOKENS, 2), jnp.int32),
            jax.ShapeDtypeStruct((_N_TOKENS, 2), jnp.float32),
        ],
        compiler_params=pltpu.CompilerParams(
            dimension_semantics=("arbitrary",),
            vmem_limit_bytes=100 * 1024 * 1024,
        ),
    )(x, W1, b1r, W2, b2r)
    return (indices, weights)


# speedup vs baseline: 1.0041x; 1.0038x over previous
import jax
import jax.numpy as jnp
from jax import lax
from jax.experimental import pallas as pl
from jax.experimental.pallas import tpu as pltpu

_INPUT_DIM = 2048
_HIDDEN_DIM = 512
_NUM_EXPERTS = 64
_N_TOKENS = 8192
_BLK = 2048


_SUB = 256


def _gating_kernel(x_ref, w1_ref, b1_ref, w2_ref, b2_ref, idx_ref, wgt_ref):
    neg_inf = jnp.float32(-jnp.inf)
    big = jnp.float32(_NUM_EXPERTS)

    for t in range(_BLK // _SUB):
        rows = pl.ds(t * _SUB, _SUB)
        h = jnp.dot(x_ref[rows, :], w1_ref[...],
                    preferred_element_type=jnp.float32)
        h = jnp.maximum(h + b1_ref[...], 0.0)
        logits = jnp.dot(h, w2_ref[...], preferred_element_type=jnp.float32)
        logits = logits + b2_ref[...]

        ids = lax.broadcasted_iota(jnp.int32, logits.shape, 1).astype(jnp.float32)

        m1 = jnp.max(logits, axis=1, keepdims=True)
        i1 = jnp.min(jnp.where(logits == m1, ids, big), axis=1, keepdims=True)
        masked = jnp.where(ids == i1, neg_inf, logits)
        m2 = jnp.max(masked, axis=1, keepdims=True)
        i2 = jnp.min(jnp.where(masked == m2, ids, big), axis=1, keepdims=True)

        e2 = jnp.exp(m2 - m1)
        w1v = 1.0 / (1.0 + e2)
        w2v = e2 * w1v

        idx_ref[rows, :] = jnp.concatenate([i1, i2], axis=1).astype(jnp.int32)
        wgt_ref[rows, :] = jnp.concatenate([w1v, w2v], axis=1)


def kernel(x, W1, b1, W2, b2):
    n_blocks = _N_TOKENS // _BLK
    b1r = b1.reshape(1, _HIDDEN_DIM)
    b2r = b2.reshape(1, _NUM_EXPERTS)

    indices, weights = pl.pallas_call(
        _gating_kernel,
        grid=(n_blocks,),
        in_specs=[
            pl.BlockSpec((_BLK, _INPUT_DIM), lambda i: (i, 0)),
            pl.BlockSpec((_INPUT_DIM, _HIDDEN_DIM), lambda i: (0, 0)),
            pl.BlockSpec((1, _HIDDEN_DIM), lambda i: (0, 0)),
            pl.BlockSpec((_HIDDEN_DIM, _NUM_EXPERTS), lambda i: (0, 0)),
            pl.BlockSpec((1, _NUM_EXPERTS), lambda i: (0, 0)),
        ],
        out_specs=[
            pl.BlockSpec((_BLK, 2), lambda i: (i, 0)),
            pl.BlockSpec((_BLK, 2), lambda i: (i, 0)),
        ],
        out_shape=[
            jax.ShapeDtypeStruct((_N_TOKENS, 2), jnp.int32),
            jax.ShapeDtypeStruct((_N_TOKENS, 2), jnp.float32),
        ],
        compiler_params=pltpu.CompilerParams(
            dimension_semantics=("arbitrary",),
            vmem_limit_bytes=100 * 1024 * 1024,
        ),
    )(x, W1, b1r, W2, b2r)
    return (indices, weights)


# speedup vs baseline: 1.0316x; 1.0275x over previous
import jax
import jax.numpy as jnp
from jax import lax
from jax.experimental import pallas as pl
from jax.experimental.pallas import tpu as pltpu

_INPUT_DIM = 2048
_HIDDEN_DIM = 512
_NUM_EXPERTS = 64
_N_TOKENS = 8192
_BLK = 2048


_SUB = 512


def _gating_kernel(x_ref, w1_ref, b1_ref, w2_ref, b2_ref, idx_ref, wgt_ref):
    neg_inf = jnp.float32(-jnp.inf)
    big = jnp.float32(_NUM_EXPERTS)

    for t in range(_BLK // _SUB):
        rows = pl.ds(t * _SUB, _SUB)
        h = jnp.dot(x_ref[rows, :], w1_ref[...],
                    preferred_element_type=jnp.float32)
        h = jnp.maximum(h + b1_ref[...], 0.0)
        logits = jnp.dot(h, w2_ref[...], preferred_element_type=jnp.float32)
        logits = logits + b2_ref[...]

        ids = lax.broadcasted_iota(jnp.int32, logits.shape, 1).astype(jnp.float32)

        m1 = jnp.max(logits, axis=1, keepdims=True)
        i1 = jnp.min(jnp.where(logits == m1, ids, big), axis=1, keepdims=True)
        masked = jnp.where(ids == i1, neg_inf, logits)
        m2 = jnp.max(masked, axis=1, keepdims=True)
        i2 = jnp.min(jnp.where(masked == m2, ids, big), axis=1, keepdims=True)

        e2 = jnp.exp(m2 - m1)
        w1v = 1.0 / (1.0 + e2)
        w2v = e2 * w1v

        idx_ref[rows, :] = jnp.concatenate([i1, i2], axis=1).astype(jnp.int32)
        wgt_ref[rows, :] = jnp.concatenate([w1v, w2v], axis=1)


def kernel(x, W1, b1, W2, b2):
    n_blocks = _N_TOKENS // _BLK
    b1r = b1.reshape(1, _HIDDEN_DIM)
    b2r = b2.reshape(1, _NUM_EXPERTS)

    indices, weights = pl.pallas_call(
        _gating_kernel,
        grid=(n_blocks,),
        in_specs=[
            pl.BlockSpec((_BLK, _INPUT_DIM), lambda i: (i, 0)),
            pl.BlockSpec((_INPUT_DIM, _HIDDEN_DIM), lambda i: (0, 0)),
            pl.BlockSpec((1, _HIDDEN_DIM), lambda i: (0, 0)),
            pl.BlockSpec((_HIDDEN_DIM, _NUM_EXPERTS), lambda i: (0, 0)),
            pl.BlockSpec((1, _NUM_EXPERTS), lambda i: (0, 0)),
        ],
        out_specs=[
            pl.BlockSpec((_BLK, 2), lambda i: (i, 0)),
            pl.BlockSpec((_BLK, 2), lambda i: (i, 0)),
        ],
        out_shape=[
            jax.ShapeDtypeStruct((_N_TOKENS, 2), jnp.int32),
            jax.ShapeDtypeStruct((_N_TOKENS, 2), jnp.float32),
        ],
        compiler_params=pltpu.CompilerParams(
            dimension_semantics=("arbitrary",),
            vmem_limit_bytes=100 * 1024 * 1024,
        ),
    )(x, W1, b1r, W2, b2r)
    return (indices, weights)
